# Initial kernel scaffold; baseline (speedup 1.0000x reference)
import functools

import jax
import jax.numpy as jnp
from jax import lax
from jax.experimental import pallas as pl
from jax.experimental.pallas import tpu as pltpu
from jax.experimental.pallas import tpu_sc as plsc

_N = 10000
_E = 320000
_DIN = 128
_DH = 16
_DO = 64

_NC = 2
_NS = 16
_NW = _NC * _NS
_C = 128
_K = 80
_EPT = _K * _C
_E_PAD = _NW * _EPT
_NPS = _N // _NS

_mesh = plsc.VectorSubcoreMesh(core_axis_name="c", subcore_axis_name="s")


def _zero_acc(zbuf, acc, s):
    @pl.loop(0, _NPS)
    def _(i):
        zbuf.at[i][...] = jnp.zeros((16,), jnp.float32)

    pltpu.sync_copy(zbuf, acc.at[pl.ds(s * _NPS, _NPS)])


@functools.partial(
    pl.kernel,
    out_type=jax.ShapeDtypeStruct((_NC, _N, _DH), jnp.float32),
    mesh=_mesh,
    scratch_types=[
        pltpu.VMEM((_K, _C), jnp.int32),
        pltpu.VMEM((_K, _C), jnp.int32),
        pltpu.VMEM((_K, _C), jnp.float32),
        pltpu.VMEM((_C, _DH), jnp.float32),
        pltpu.VMEM((_NPS, _DH), jnp.float32),
        pltpu.VMEM_SHARED((_N, _DH), jnp.float32),
    ],
)
def _agg_sc(h_hbm, src_hbm, dst_hbm, ew_hbm, out_hbm,
            srcv, dstv, ewv, rows, zbuf, acc):
    c = lax.axis_index("c")
    s = lax.axis_index("s")
    wid = c * _NS + s

    _zero_acc(zbuf, acc, s)
    plsc.subcore_barrier()

    pltpu.sync_copy(src_hbm.at[wid], srcv)
    pltpu.sync_copy(dst_hbm.at[wid], dstv)
    pltpu.sync_copy(ew_hbm.at[wid], ewv)

    @pl.loop(0, _K)
    def _(j):
        pltpu.sync_copy(h_hbm.at[srcv.at[j]], rows)

        @pl.loop(0, _C // 16)
        def _(g):
            base = g * 16
            for t in range(16):
                e = base + t
                idx = jnp.zeros((16,), jnp.int32) + e
                splat = plsc.load_gather(ewv.at[j], [idx])
                rows.at[e][...] = rows.at[e][...] * splat

        pltpu.sync_copy(rows, acc.at[dstv.at[j]], add=True)

    plsc.subcore_barrier()
    pltpu.sync_copy(acc.at[pl.ds(s * _NPS, _NPS)],
                    out_hbm.at[c, pl.ds(s * _NPS, _NPS)])


@functools.partial(
    pl.kernel,
    out_type=jax.ShapeDtypeStruct((_NC, _N, _DH), jnp.float32),
    mesh=_mesh,
    scratch_types=[
        pltpu.VMEM((_K, _C), jnp.int32),
        pltpu.VMEM((_K, _C), jnp.float32),
        pltpu.VMEM((_C, _DH), jnp.float32),
        pltpu.VMEM((_NPS, _DH), jnp.float32),
        pltpu.VMEM_SHARED((_N, _DH), jnp.float32),
    ],
)
def _deg_sc(dst_hbm, ew_hbm, out_hbm, dstv, ewv, rows, zbuf, acc):
    c = lax.axis_index("c")
    s = lax.axis_index("s")
    wid = c * _NS + s

    _zero_acc(zbuf, acc, s)
    plsc.subcore_barrier()

    pltpu.sync_copy(dst_hbm.at[wid], dstv)
    pltpu.sync_copy(ew_hbm.at[wid], ewv)

    @pl.loop(0, _K)
    def _(j):
        @pl.loop(0, _C // 16)
        def _(g):
            base = g * 16
            for t in range(16):
                e = base + t
                idx = jnp.zeros((16,), jnp.int32) + e
                rows.at[e][...] = plsc.load_gather(ewv.at[j], [idx])

        pltpu.sync_copy(rows, acc.at[dstv.at[j]], add=True)

    plsc.subcore_barrier()
    pltpu.sync_copy(acc.at[pl.ds(s * _NPS, _NPS)],
                    out_hbm.at[c, pl.ds(s * _NPS, _NPS)])


def _mm1_body(x_ref, w_ref, o_ref):
    o_ref[...] = jnp.dot(x_ref[...], w_ref[...],
                         preferred_element_type=jnp.float32)


_mm1 = pl.pallas_call(
    _mm1_body,
    out_shape=jax.ShapeDtypeStruct((_N, _DH), jnp.float32),
)


def _prep_body(degp_ref, h1_ref, h1d_ref, sl1_ref, dinv_ref):
    degp = degp_ref[...]
    deg = degp[0, :, 0:1] + degp[1, :, 0:1] + 1.0
    dv = lax.rsqrt(deg)
    h1 = h1_ref[...]
    h1d = h1 * dv
    h1d_ref[...] = h1d
    sl1_ref[...] = h1d * dv
    dinv_ref[...] = dv


_prep = pl.pallas_call(
    _prep_body,
    out_shape=(
        jax.ShapeDtypeStruct((_N, _DH), jnp.float32),
        jax.ShapeDtypeStruct((_N, _DH), jnp.float32),
        jax.ShapeDtypeStruct((_N, 1), jnp.float32),
    ),
)


def _mid_body(aggp_ref, sl1_ref, dinv_ref, b1_ref, z1d_ref, slz_ref):
    aggp = aggp_ref[...]
    dv = dinv_ref[...]
    t = dv * (aggp[0] + aggp[1]) + sl1_ref[...] + b1_ref[...]
    z1 = jnp.maximum(t, 0.0)
    z1d = z1 * dv
    z1d_ref[...] = z1d
    slz_ref[...] = z1d * dv


_mid = pl.pallas_call(
    _mid_body,
    out_shape=(
        jax.ShapeDtypeStruct((_N, _DH), jnp.float32),
        jax.ShapeDtypeStruct((_N, _DH), jnp.float32),
    ),
)


def _fin_body(aggp_ref, slz_ref, dinv_ref, w2_ref, b2_ref, o_ref):
    aggp = aggp_ref[...]
    pre = dinv_ref[...] * (aggp[0] + aggp[1]) + slz_ref[...]
    o_ref[...] = jnp.dot(pre, w2_ref[...],
                         preferred_element_type=jnp.float32) + b2_ref[...]


_fin = pl.pallas_call(
    _fin_body,
    out_shape=jax.ShapeDtypeStruct((_N, _DO), jnp.float32),
)


def kernel(x, edge_index, edge_weight, W1, b1, W2, b2):
    src = edge_index[0].astype(jnp.int32)
    dst = edge_index[1].astype(jnp.int32)
    ew = edge_weight.astype(jnp.float32)
    pad = _E_PAD - _E
    src3 = jnp.pad(src, (0, pad)).reshape(_NW, _K, _C)
    dst3 = jnp.pad(dst, (0, pad)).reshape(_NW, _K, _C)
    ew3 = jnp.pad(ew, (0, pad)).reshape(_NW, _K, _C)

    h1 = _mm1(x, W1)
    degp = _deg_sc(dst3, ew3)
    h1d, sl1, dinv = _prep(degp, h1)
    aggp1 = _agg_sc(h1d, src3, dst3, ew3)
    z1d, slz = _mid(aggp1, sl1, dinv, b1.reshape(1, _DH))
    aggp2 = _agg_sc(z1d, src3, dst3, ew3)
    return _fin(aggp2, slz, dinv, W2, b2.reshape(1, _DO))


# speedup vs baseline: 19.7872x; 19.7872x over previous
import functools

import jax
import jax.numpy as jnp
from jax import lax
from jax.experimental import pallas as pl
from jax.experimental.pallas import tpu as pltpu
from jax.experimental.pallas import tpu_sc as plsc

_N = 10000
_E = 320000
_DIN = 128
_DH = 16
_DO = 64

_NC = 2
_NS = 16
_NW = _NC * _NS
_C = 128
_K = 80
_EPT = _K * _C
_E_PAD = _NW * _EPT
_NP = 10240
_NPS = _NP // _NS

_mesh = plsc.VectorSubcoreMesh(core_axis_name="c", subcore_axis_name="s")

_sc_params = pltpu.CompilerParams(needs_layout_passes=False,
                                  use_tc_tiling_on_sc=False)


def _zero_acc(zbuf, acc, s):
    @pl.loop(0, _NPS)
    def _(i):
        zbuf.at[i][...] = jnp.zeros((16,), jnp.float32)

    pltpu.sync_copy(zbuf, acc.at[pl.ds(s * _NPS, _NPS)])


@functools.partial(
    pl.kernel,
    out_type=jax.ShapeDtypeStruct((_NC, _NP, _DH), jnp.float32),
    mesh=_mesh,
    compiler_params=_sc_params,
    scratch_types=[
        pltpu.VMEM((_K, _C), jnp.int32),
        pltpu.VMEM((_K, _C), jnp.int32),
        pltpu.VMEM((_K, _C), jnp.float32),
        pltpu.VMEM((_C, _DH), jnp.float32),
        pltpu.VMEM((_NPS, _DH), jnp.float32),
        pltpu.VMEM_SHARED((_NP, _DH), jnp.float32),
    ],
)
def _agg_sc(h_hbm, src_hbm, dst_hbm, ew_hbm, out_hbm,
            srcv, dstv, ewv, rows, zbuf, acc):
    c = lax.axis_index("c")
    s = lax.axis_index("s")
    wid = c * _NS + s

    _zero_acc(zbuf, acc, s)
    plsc.subcore_barrier()

    pltpu.sync_copy(src_hbm.at[wid], srcv)
    pltpu.sync_copy(dst_hbm.at[wid], dstv)
    pltpu.sync_copy(ew_hbm.at[wid], ewv)

    @pl.loop(0, _K)
    def _(j):
        pltpu.sync_copy(h_hbm.at[srcv.at[j]], rows)

        @pl.loop(0, _C // 16)
        def _(g):
            base = g * 16
            for t in range(16):
                e = base + t
                idx = jnp.zeros((16,), jnp.int32) + e
                splat = plsc.load_gather(ewv.at[j], [idx])
                rows.at[e][...] = rows.at[e][...] * splat

        pltpu.sync_copy(rows, acc.at[dstv.at[j]], add=True)

    plsc.subcore_barrier()
    pltpu.sync_copy(acc.at[pl.ds(s * _NPS, _NPS)],
                    out_hbm.at[c, pl.ds(s * _NPS, _NPS)])


@functools.partial(
    pl.kernel,
    out_type=jax.ShapeDtypeStruct((_NC, _NP, _DH), jnp.float32),
    mesh=_mesh,
    compiler_params=_sc_params,
    scratch_types=[
        pltpu.VMEM((_K, _C), jnp.int32),
        pltpu.VMEM((_K, _C), jnp.float32),
        pltpu.VMEM((_C, _DH), jnp.float32),
        pltpu.VMEM((_NPS, _DH), jnp.float32),
        pltpu.VMEM_SHARED((_NP, _DH), jnp.float32),
    ],
)
def _deg_sc(dst_hbm, ew_hbm, out_hbm, dstv, ewv, rows, zbuf, acc):
    c = lax.axis_index("c")
    s = lax.axis_index("s")
    wid = c * _NS + s

    _zero_acc(zbuf, acc, s)
    plsc.subcore_barrier()

    pltpu.sync_copy(dst_hbm.at[wid], dstv)
    pltpu.sync_copy(ew_hbm.at[wid], ewv)

    @pl.loop(0, _K)
    def _(j):
        @pl.loop(0, _C // 16)
        def _(g):
            base = g * 16
            for t in range(16):
                e = base + t
                idx = jnp.zeros((16,), jnp.int32) + e
                rows.at[e][...] = plsc.load_gather(ewv.at[j], [idx])

        pltpu.sync_copy(rows, acc.at[dstv.at[j]], add=True)

    plsc.subcore_barrier()
    pltpu.sync_copy(acc.at[pl.ds(s * _NPS, _NPS)],
                    out_hbm.at[c, pl.ds(s * _NPS, _NPS)])


def _mm1_body(x_ref, w_ref, o_ref):
    o_ref[...] = jnp.dot(x_ref[...], w_ref[...],
                         preferred_element_type=jnp.float32)


_mm1 = pl.pallas_call(
    _mm1_body,
    out_shape=jax.ShapeDtypeStruct((_NP, _DH), jnp.float32),
)


def _prep_body(degp_ref, h1_ref, h1d_ref, sl1_ref, dinv_ref):
    degp = degp_ref[...]
    deg = degp[0, :, 0:1] + degp[1, :, 0:1] + 1.0
    dv = lax.rsqrt(deg)
    h1 = h1_ref[...]
    h1d = h1 * dv
    h1d_ref[...] = h1d
    sl1_ref[...] = h1d * dv
    dinv_ref[...] = dv


_prep = pl.pallas_call(
    _prep_body,
    out_shape=(
        jax.ShapeDtypeStruct((_NP, _DH), jnp.float32),
        jax.ShapeDtypeStruct((_NP, _DH), jnp.float32),
        jax.ShapeDtypeStruct((_NP, 1), jnp.float32),
    ),
)


def _mid_body(aggp_ref, sl1_ref, dinv_ref, b1_ref, z1d_ref, slz_ref):
    aggp = aggp_ref[...]
    dv = dinv_ref[...]
    t = dv * (aggp[0] + aggp[1]) + sl1_ref[...] + b1_ref[...]
    z1 = jnp.maximum(t, 0.0)
    z1d = z1 * dv
    z1d_ref[...] = z1d
    slz_ref[...] = z1d * dv


_mid = pl.pallas_call(
    _mid_body,
    out_shape=(
        jax.ShapeDtypeStruct((_NP, _DH), jnp.float32),
        jax.ShapeDtypeStruct((_NP, _DH), jnp.float32),
    ),
)


def _fin_body(aggp_ref, slz_ref, dinv_ref, w2_ref, b2_ref, o_ref):
    aggp = aggp_ref[...]
    pre = dinv_ref[...] * (aggp[0] + aggp[1]) + slz_ref[...]
    o_ref[...] = jnp.dot(pre, w2_ref[...],
                         preferred_element_type=jnp.float32) + b2_ref[...]


_fin = pl.pallas_call(
    _fin_body,
    out_shape=jax.ShapeDtypeStruct((_NP, _DO), jnp.float32),
)


def kernel(x, edge_index, edge_weight, W1, b1, W2, b2):
    src = edge_index[0].astype(jnp.int32)
    dst = edge_index[1].astype(jnp.int32)
    ew = edge_weight.astype(jnp.float32)
    pad = _E_PAD - _E
    src3 = jnp.pad(src, (0, pad)).reshape(_NW, _K, _C)
    dst3 = jnp.pad(dst, (0, pad)).reshape(_NW, _K, _C)
    ew3 = jnp.pad(ew, (0, pad)).reshape(_NW, _K, _C)

    xp = jnp.pad(x, ((0, _NP - _N), (0, 0)))
    h1 = _mm1(xp, W1)
    degp = _deg_sc(dst3, ew3)
    h1d, sl1, dinv = _prep(degp, h1)
    aggp1 = _agg_sc(h1d, src3, dst3, ew3)
    z1d, slz = _mid(aggp1, sl1, dinv, b1.reshape(1, _DH))
    aggp2 = _agg_sc(z1d, src3, dst3, ew3)
    out = _fin(aggp2, slz, dinv, W2, b2.reshape(1, _DO))
    return out[:_N]


# speedup vs baseline: 23.5467x; 1.1900x over previous
import functools

import jax
import jax.numpy as jnp
from jax import lax
from jax.experimental import pallas as pl
from jax.experimental.pallas import tpu as pltpu
from jax.experimental.pallas import tpu_sc as plsc

_N = 10000
_E = 320000
_DIN = 128
_DH = 16
_DO = 64

_NC = 2
_NS = 16
_NW = _NC * _NS
_C = 128
_K = 80
_EPT = _K * _C
_E_PAD = _NW * _EPT
_NP = 10240
_NPS = _NP // _NS

_mesh = plsc.VectorSubcoreMesh(core_axis_name="c", subcore_axis_name="s")

_sc_params = pltpu.CompilerParams(needs_layout_passes=False,
                                  use_tc_tiling_on_sc=False)


def _zero_acc(zbuf, acc, s):
    @pl.loop(0, _NPS)
    def _(i):
        zbuf.at[i][...] = jnp.zeros((16,), jnp.float32)

    pltpu.sync_copy(zbuf, acc.at[pl.ds(s * _NPS, _NPS)])


_NBUF = 4


def _scale_rows(rows, ewrow):
    @pl.loop(0, _C // 16)
    def _(g):
        base = g * 16
        for t in range(16):
            e = base + t
            idx = jnp.zeros((16,), jnp.int32) + e
            splat = plsc.load_gather(ewrow, [idx])
            rows.at[e][...] = rows.at[e][...] * splat


def _fill_rows(rows, ewrow):
    @pl.loop(0, _C // 16)
    def _(g):
        base = g * 16
        for t in range(16):
            e = base + t
            idx = jnp.zeros((16,), jnp.int32) + e
            rows.at[e][...] = plsc.load_gather(ewrow, [idx])


@functools.partial(
    pl.kernel,
    out_type=jax.ShapeDtypeStruct((_NC, _NP, _DH), jnp.float32),
    mesh=_mesh,
    compiler_params=_sc_params,
    scratch_types=(
        [
            pltpu.VMEM((_K, _C), jnp.int32),
            pltpu.VMEM((_K, _C), jnp.int32),
            pltpu.VMEM((_K, _C), jnp.float32),
        ]
        + [pltpu.VMEM((_C, _DH), jnp.float32) for _ in range(_NBUF)]
        + [pltpu.VMEM((_NPS, _DH), jnp.float32),
           pltpu.VMEM_SHARED((_NP, _DH), jnp.float32)]
        + [pltpu.SemaphoreType.DMA for _ in range(2 * _NBUF)]
    ),
)
def _agg_sc(h_hbm, src_hbm, dst_hbm, ew_hbm, out_hbm,
            srcv, dstv, ewv, *rest):
    rows = rest[:_NBUF]
    zbuf, acc = rest[_NBUF], rest[_NBUF + 1]
    gsem = rest[_NBUF + 2:_NBUF + 2 + _NBUF]
    ssem = rest[_NBUF + 2 + _NBUF:]
    c = lax.axis_index("c")
    s = lax.axis_index("s")
    wid = c * _NS + s

    _zero_acc(zbuf, acc, s)
    plsc.subcore_barrier()

    pltpu.sync_copy(src_hbm.at[wid], srcv)
    pltpu.sync_copy(dst_hbm.at[wid], dstv)
    pltpu.sync_copy(ew_hbm.at[wid], ewv)

    @pl.loop(0, _K // _NBUF)
    def _(i):
        j0 = i * _NBUF
        hg = [pltpu.async_copy(h_hbm.at[srcv.at[j0 + b]], rows[b], gsem[b])
              for b in range(_NBUF)]
        hs = []
        for b in range(_NBUF):
            hg[b].wait()
            _scale_rows(rows[b], ewv.at[j0 + b])
            hs.append(pltpu.async_copy(rows[b], acc.at[dstv.at[j0 + b]],
                                       ssem[b], add=True))
        for h in hs:
            h.wait()

    plsc.subcore_barrier()
    pltpu.sync_copy(acc.at[pl.ds(s * _NPS, _NPS)],
                    out_hbm.at[c, pl.ds(s * _NPS, _NPS)])


@functools.partial(
    pl.kernel,
    out_type=jax.ShapeDtypeStruct((_NC, _NP, _DH), jnp.float32),
    mesh=_mesh,
    compiler_params=_sc_params,
    scratch_types=(
        [
            pltpu.VMEM((_K, _C), jnp.int32),
            pltpu.VMEM((_K, _C), jnp.float32),
        ]
        + [pltpu.VMEM((_C, _DH), jnp.float32) for _ in range(_NBUF)]
        + [pltpu.VMEM((_NPS, _DH), jnp.float32),
           pltpu.VMEM_SHARED((_NP, _DH), jnp.float32)]
        + [pltpu.SemaphoreType.DMA for _ in range(_NBUF)]
    ),
)
def _deg_sc(dst_hbm, ew_hbm, out_hbm, dstv, ewv, *rest):
    rows = rest[:_NBUF]
    zbuf, acc = rest[_NBUF], rest[_NBUF + 1]
    ssem = rest[_NBUF + 2:]
    c = lax.axis_index("c")
    s = lax.axis_index("s")
    wid = c * _NS + s

    _zero_acc(zbuf, acc, s)
    plsc.subcore_barrier()

    pltpu.sync_copy(dst_hbm.at[wid], dstv)
    pltpu.sync_copy(ew_hbm.at[wid], ewv)

    @pl.loop(0, _K // _NBUF)
    def _(i):
        j0 = i * _NBUF
        hs = []
        for b in range(_NBUF):
            _fill_rows(rows[b], ewv.at[j0 + b])
            hs.append(pltpu.async_copy(rows[b], acc.at[dstv.at[j0 + b]],
                                       ssem[b], add=True))
        for h in hs:
            h.wait()

    plsc.subcore_barrier()
    pltpu.sync_copy(acc.at[pl.ds(s * _NPS, _NPS)],
                    out_hbm.at[c, pl.ds(s * _NPS, _NPS)])


def _mm1_body(x_ref, w_ref, o_ref):
    o_ref[...] = jnp.dot(x_ref[...], w_ref[...],
                         preferred_element_type=jnp.float32)


_mm1 = pl.pallas_call(
    _mm1_body,
    out_shape=jax.ShapeDtypeStruct((_NP, _DH), jnp.float32),
)


def _prep_body(degp_ref, h1_ref, h1d_ref, sl1_ref, dinv_ref):
    degp = degp_ref[...]
    deg = degp[0, :, 0:1] + degp[1, :, 0:1] + 1.0
    dv = lax.rsqrt(deg)
    h1 = h1_ref[...]
    h1d = h1 * dv
    h1d_ref[...] = h1d
    sl1_ref[...] = h1d * dv
    dinv_ref[...] = dv


_prep = pl.pallas_call(
    _prep_body,
    out_shape=(
        jax.ShapeDtypeStruct((_NP, _DH), jnp.float32),
        jax.ShapeDtypeStruct((_NP, _DH), jnp.float32),
        jax.ShapeDtypeStruct((_NP, 1), jnp.float32),
    ),
)


def _mid_body(aggp_ref, sl1_ref, dinv_ref, b1_ref, z1d_ref, slz_ref):
    aggp = aggp_ref[...]
    dv = dinv_ref[...]
    t = dv * (aggp[0] + aggp[1]) + sl1_ref[...] + b1_ref[...]
    z1 = jnp.maximum(t, 0.0)
    z1d = z1 * dv
    z1d_ref[...] = z1d
    slz_ref[...] = z1d * dv


_mid = pl.pallas_call(
    _mid_body,
    out_shape=(
        jax.ShapeDtypeStruct((_NP, _DH), jnp.float32),
        jax.ShapeDtypeStruct((_NP, _DH), jnp.float32),
    ),
)


def _fin_body(aggp_ref, slz_ref, dinv_ref, w2_ref, b2_ref, o_ref):
    aggp = aggp_ref[...]
    pre = dinv_ref[...] * (aggp[0] + aggp[1]) + slz_ref[...]
    o_ref[...] = jnp.dot(pre, w2_ref[...],
                         preferred_element_type=jnp.float32) + b2_ref[...]


_fin = pl.pallas_call(
    _fin_body,
    out_shape=jax.ShapeDtypeStruct((_NP, _DO), jnp.float32),
)


def kernel(x, edge_index, edge_weight, W1, b1, W2, b2):
    src = edge_index[0].astype(jnp.int32)
    dst = edge_index[1].astype(jnp.int32)
    ew = edge_weight.astype(jnp.float32)
    pad = _E_PAD - _E
    src3 = jnp.pad(src, (0, pad)).reshape(_NW, _K, _C)
    dst3 = jnp.pad(dst, (0, pad)).reshape(_NW, _K, _C)
    ew3 = jnp.pad(ew, (0, pad)).reshape(_NW, _K, _C)

    xp = jnp.pad(x, ((0, _NP - _N), (0, 0)))
    h1 = _mm1(xp, W1)
    degp = _deg_sc(dst3, ew3)
    h1d, sl1, dinv = _prep(degp, h1)
    aggp1 = _agg_sc(h1d, src3, dst3, ew3)
    z1d, slz = _mid(aggp1, sl1, dinv, b1.reshape(1, _DH))
    aggp2 = _agg_sc(z1d, src3, dst3, ew3)
    out = _fin(aggp2, slz, dinv, W2, b2.reshape(1, _DO))
    return out[:_N]


# speedup vs baseline: 34.4717x; 1.4640x over previous
import functools

import jax
import jax.numpy as jnp
from jax import lax
from jax.experimental import pallas as pl
from jax.experimental.pallas import tpu as pltpu
from jax.experimental.pallas import tpu_sc as plsc

_N = 10000
_E = 320000
_DIN = 128
_DH = 16
_DO = 64

_NC = 2
_NS = 16
_NW = _NC * _NS
_C = 128
_K = 80
_EPT = _K * _C
_E_PAD = _NW * _EPT
_NP = 10240
_NPS = _NP // _NS

_mesh = plsc.VectorSubcoreMesh(core_axis_name="c", subcore_axis_name="s")

_sc_params = pltpu.CompilerParams(needs_layout_passes=False,
                                  use_tc_tiling_on_sc=False)


def _zero_acc(zbuf, acc, s):
    @pl.loop(0, _NPS)
    def _(i):
        zbuf.at[i][...] = jnp.zeros((16,), jnp.float32)

    pltpu.sync_copy(zbuf, acc.at[pl.ds(s * _NPS, _NPS)])


_NBUF = 8


def _scale_rows(rows, ewrow):
    @pl.loop(0, _C // 16)
    def _(g):
        base = g * 16
        ev = ewrow[pl.ds(base, 16)]
        for t in range(16):
            e = base + t
            splat = jnp.take(ev, jnp.full((16,), t, jnp.int32))
            rows.at[e][...] = rows.at[e][...] * splat


def _fill_rows(rows, ewrow):
    @pl.loop(0, _C // 16)
    def _(g):
        base = g * 16
        ev = ewrow[pl.ds(base, 16)]
        for t in range(16):
            e = base + t
            rows.at[e][...] = jnp.take(ev, jnp.full((16,), t, jnp.int32))


@functools.partial(
    pl.kernel,
    out_type=jax.ShapeDtypeStruct((_NC, _NP, _DH), jnp.float32),
    mesh=_mesh,
    compiler_params=_sc_params,
    scratch_types=(
        [
            pltpu.VMEM((_K, _C), jnp.int32),
            pltpu.VMEM((_K, _C), jnp.int32),
            pltpu.VMEM((_K, _C), jnp.float32),
        ]
        + [pltpu.VMEM((_C, _DH), jnp.float32) for _ in range(_NBUF)]
        + [pltpu.VMEM((_NPS, _DH), jnp.float32),
           pltpu.VMEM_SHARED((_NP, _DH), jnp.float32)]
        + [pltpu.SemaphoreType.DMA for _ in range(2 * _NBUF)]
    ),
)
def _agg_sc(h_hbm, src_hbm, dst_hbm, ew_hbm, out_hbm,
            srcv, dstv, ewv, *rest):
    rows = rest[:_NBUF]
    zbuf, acc = rest[_NBUF], rest[_NBUF + 1]
    gsem = rest[_NBUF + 2:_NBUF + 2 + _NBUF]
    ssem = rest[_NBUF + 2 + _NBUF:]
    c = lax.axis_index("c")
    s = lax.axis_index("s")
    wid = c * _NS + s

    _zero_acc(zbuf, acc, s)
    plsc.subcore_barrier()

    pltpu.sync_copy(src_hbm.at[wid], srcv)
    pltpu.sync_copy(dst_hbm.at[wid], dstv)
    pltpu.sync_copy(ew_hbm.at[wid], ewv)

    @pl.loop(0, _K // _NBUF)
    def _(i):
        j0 = i * _NBUF
        hg = [pltpu.async_copy(h_hbm.at[srcv.at[j0 + b]], rows[b], gsem[b])
              for b in range(_NBUF)]
        hs = []
        for b in range(_NBUF):
            hg[b].wait()
            _scale_rows(rows[b], ewv.at[j0 + b])
            hs.append(pltpu.async_copy(rows[b], acc.at[dstv.at[j0 + b]],
                                       ssem[b], add=True))
        for h in hs:
            h.wait()

    plsc.subcore_barrier()
    pltpu.sync_copy(acc.at[pl.ds(s * _NPS, _NPS)],
                    out_hbm.at[c, pl.ds(s * _NPS, _NPS)])


@functools.partial(
    pl.kernel,
    out_type=jax.ShapeDtypeStruct((_NC, _NP, _DH), jnp.float32),
    mesh=_mesh,
    compiler_params=_sc_params,
    scratch_types=(
        [
            pltpu.VMEM((_K, _C), jnp.int32),
            pltpu.VMEM((_K, _C), jnp.float32),
        ]
        + [pltpu.VMEM((_C, _DH), jnp.float32) for _ in range(_NBUF)]
        + [pltpu.VMEM((_NPS, _DH), jnp.float32),
           pltpu.VMEM_SHARED((_NP, _DH), jnp.float32)]
        + [pltpu.SemaphoreType.DMA for _ in range(_NBUF)]
    ),
)
def _deg_sc(dst_hbm, ew_hbm, out_hbm, dstv, ewv, *rest):
    rows = rest[:_NBUF]
    zbuf, acc = rest[_NBUF], rest[_NBUF + 1]
    ssem = rest[_NBUF + 2:]
    c = lax.axis_index("c")
    s = lax.axis_index("s")
    wid = c * _NS + s

    _zero_acc(zbuf, acc, s)
    plsc.subcore_barrier()

    pltpu.sync_copy(dst_hbm.at[wid], dstv)
    pltpu.sync_copy(ew_hbm.at[wid], ewv)

    @pl.loop(0, _K // _NBUF)
    def _(i):
        j0 = i * _NBUF
        hs = []
        for b in range(_NBUF):
            _fill_rows(rows[b], ewv.at[j0 + b])
            hs.append(pltpu.async_copy(rows[b], acc.at[dstv.at[j0 + b]],
                                       ssem[b], add=True))
        for h in hs:
            h.wait()

    plsc.subcore_barrier()
    pltpu.sync_copy(acc.at[pl.ds(s * _NPS, _NPS)],
                    out_hbm.at[c, pl.ds(s * _NPS, _NPS)])


def _mm1_body(x_ref, w_ref, o_ref):
    o_ref[...] = jnp.dot(x_ref[...], w_ref[...],
                         preferred_element_type=jnp.float32)


_mm1 = pl.pallas_call(
    _mm1_body,
    out_shape=jax.ShapeDtypeStruct((_NP, _DH), jnp.float32),
)


def _prep_body(degp_ref, h1_ref, h1d_ref, sl1_ref, dinv_ref):
    degp = degp_ref[...]
    deg = degp[0, :, 0:1] + degp[1, :, 0:1] + 1.0
    dv = lax.rsqrt(deg)
    h1 = h1_ref[...]
    h1d = h1 * dv
    h1d_ref[...] = h1d
    sl1_ref[...] = h1d * dv
    dinv_ref[...] = dv


_prep = pl.pallas_call(
    _prep_body,
    out_shape=(
        jax.ShapeDtypeStruct((_NP, _DH), jnp.float32),
        jax.ShapeDtypeStruct((_NP, _DH), jnp.float32),
        jax.ShapeDtypeStruct((_NP, 1), jnp.float32),
    ),
)


def _mid_body(aggp_ref, sl1_ref, dinv_ref, b1_ref, z1d_ref, slz_ref):
    aggp = aggp_ref[...]
    dv = dinv_ref[...]
    t = dv * (aggp[0] + aggp[1]) + sl1_ref[...] + b1_ref[...]
    z1 = jnp.maximum(t, 0.0)
    z1d = z1 * dv
    z1d_ref[...] = z1d
    slz_ref[...] = z1d * dv


_mid = pl.pallas_call(
    _mid_body,
    out_shape=(
        jax.ShapeDtypeStruct((_NP, _DH), jnp.float32),
        jax.ShapeDtypeStruct((_NP, _DH), jnp.float32),
    ),
)


def _fin_body(aggp_ref, slz_ref, dinv_ref, w2_ref, b2_ref, o_ref):
    aggp = aggp_ref[...]
    pre = dinv_ref[...] * (aggp[0] + aggp[1]) + slz_ref[...]
    o_ref[...] = jnp.dot(pre, w2_ref[...],
                         preferred_element_type=jnp.float32) + b2_ref[...]


_fin = pl.pallas_call(
    _fin_body,
    out_shape=jax.ShapeDtypeStruct((_NP, _DO), jnp.float32),
)


def kernel(x, edge_index, edge_weight, W1, b1, W2, b2):
    src = edge_index[0].astype(jnp.int32)
    dst = edge_index[1].astype(jnp.int32)
    ew = edge_weight.astype(jnp.float32)
    pad = _E_PAD - _E
    src3 = jnp.pad(src, (0, pad)).reshape(_NW, _K, _C)
    dst3 = jnp.pad(dst, (0, pad)).reshape(_NW, _K, _C)
    ew3 = jnp.pad(ew, (0, pad)).reshape(_NW, _K, _C)

    xp = jnp.pad(x, ((0, _NP - _N), (0, 0)))
    h1 = _mm1(xp, W1)
    degp = _deg_sc(dst3, ew3)
    h1d, sl1, dinv = _prep(degp, h1)
    aggp1 = _agg_sc(h1d, src3, dst3, ew3)
    z1d, slz = _mid(aggp1, sl1, dinv, b1.reshape(1, _DH))
    aggp2 = _agg_sc(z1d, src3, dst3, ew3)
    out = _fin(aggp2, slz, dinv, W2, b2.reshape(1, _DO))
    return out[:_N]


# speedup vs baseline: 35.8173x; 1.0390x over previous
import functools

import jax
import jax.numpy as jnp
from jax import lax
from jax.experimental import pallas as pl
from jax.experimental.pallas import tpu as pltpu
from jax.experimental.pallas import tpu_sc as plsc

_N = 10000
_E = 320000
_DIN = 128
_DH = 16
_DO = 64

_NC = 2
_NS = 16
_NW = _NC * _NS
_C = 128
_K = 80
_EPT = _K * _C
_E_PAD = _NW * _EPT
_NP = 10240
_NPS = _NP // _NS

_mesh = plsc.VectorSubcoreMesh(core_axis_name="c", subcore_axis_name="s")

_sc_params = pltpu.CompilerParams(needs_layout_passes=False,
                                  use_tc_tiling_on_sc=False)


def _zero_acc(zbuf, acc, s):
    @pl.loop(0, _NPS)
    def _(i):
        zbuf.at[i][...] = jnp.zeros((16,), jnp.float32)

    pltpu.sync_copy(zbuf, acc.at[pl.ds(s * _NPS, _NPS)])


_NBUF = 8
_NCHUNK = _E_PAD // _C
_KC0 = 104
_KC1 = 56
_KMAX = max(_KC0, _KC1)


def _scale_rows(rows, ewrow):
    @pl.loop(0, _C // 16)
    def _(g):
        base = g * 16
        ev = ewrow[pl.ds(base, 16)]
        for t in range(16):
            e = base + t
            splat = jnp.take(ev, jnp.full((16,), t, jnp.int32))
            rows.at[e][...] = rows.at[e][...] * splat


def _fill_rows(rows, ewrow):
    @pl.loop(0, _C // 16)
    def _(g):
        base = g * 16
        ev = ewrow[pl.ds(base, 16)]
        for t in range(16):
            e = base + t
            rows.at[e][...] = jnp.take(ev, jnp.full((16,), t, jnp.int32))


@functools.partial(
    pl.kernel,
    out_type=jax.ShapeDtypeStruct((_NC, _NP, _DH), jnp.float32),
    mesh=_mesh,
    compiler_params=_sc_params,
    scratch_types=(
        [
            pltpu.VMEM((_KMAX, _C), jnp.int32),
            pltpu.VMEM((_KMAX, _C), jnp.int32),
            pltpu.VMEM((_KMAX, _C), jnp.float32),
        ]
        + [pltpu.VMEM((_C, _DH), jnp.float32) for _ in range(_NBUF)]
        + [pltpu.VMEM((_NPS, _DH), jnp.float32),
           pltpu.VMEM_SHARED((_NP, _DH), jnp.float32)]
        + [pltpu.SemaphoreType.DMA for _ in range(2 * _NBUF)]
    ),
)
def _agg_sc(h_hbm, src_hbm, dst_hbm, ew_hbm, out_hbm,
            srcv, dstv, ewv, *rest):
    rows = rest[:_NBUF]
    zbuf, acc = rest[_NBUF], rest[_NBUF + 1]
    gsem = rest[_NBUF + 2:_NBUF + 2 + _NBUF]
    ssem = rest[_NBUF + 2 + _NBUF:]
    c = lax.axis_index("c")
    s = lax.axis_index("s")

    _zero_acc(zbuf, acc, s)
    plsc.subcore_barrier()

    def _edges(off, nk):
        pltpu.sync_copy(src_hbm.at[pl.ds(off, nk)], srcv.at[pl.ds(0, nk)])
        pltpu.sync_copy(dst_hbm.at[pl.ds(off, nk)], dstv.at[pl.ds(0, nk)])
        pltpu.sync_copy(ew_hbm.at[pl.ds(off, nk)], ewv.at[pl.ds(0, nk)])

        @pl.loop(0, nk // _NBUF)
        def _(i):
            j0 = i * _NBUF
            hg = [pltpu.async_copy(h_hbm.at[srcv.at[j0 + b]], rows[b],
                                   gsem[b])
                  for b in range(_NBUF)]
            hs = []
            for b in range(_NBUF):
                hg[b].wait()
                _scale_rows(rows[b], ewv.at[j0 + b])
                hs.append(pltpu.async_copy(rows[b], acc.at[dstv.at[j0 + b]],
                                           ssem[b], add=True))
            for h in hs:
                h.wait()

    @pl.when(c == 0)
    def _():
        _edges(s * _KC0, _KC0)

    @pl.when(c == 1)
    def _():
        _edges(_NS * _KC0 + s * _KC1, _KC1)

    plsc.subcore_barrier()
    pltpu.sync_copy(acc.at[pl.ds(s * _NPS, _NPS)],
                    out_hbm.at[c, pl.ds(s * _NPS, _NPS)])


@functools.partial(
    pl.kernel,
    out_type=jax.ShapeDtypeStruct((_NC, _NP, _DH), jnp.float32),
    mesh=_mesh,
    compiler_params=_sc_params,
    scratch_types=(
        [
            pltpu.VMEM((_K, _C), jnp.int32),
            pltpu.VMEM((_K, _C), jnp.float32),
        ]
        + [pltpu.VMEM((_C, _DH), jnp.float32) for _ in range(_NBUF)]
        + [pltpu.VMEM((_NPS, _DH), jnp.float32),
           pltpu.VMEM_SHARED((_NP, _DH), jnp.float32)]
        + [pltpu.SemaphoreType.DMA for _ in range(_NBUF)]
    ),
)
def _deg_sc(dst_hbm, ew_hbm, out_hbm, dstv, ewv, *rest):
    rows = rest[:_NBUF]
    zbuf, acc = rest[_NBUF], rest[_NBUF + 1]
    ssem = rest[_NBUF + 2:]
    c = lax.axis_index("c")
    s = lax.axis_index("s")
    wid = c * _NS + s

    _zero_acc(zbuf, acc, s)
    plsc.subcore_barrier()

    pltpu.sync_copy(dst_hbm.at[pl.ds(wid * _K, _K)], dstv)
    pltpu.sync_copy(ew_hbm.at[pl.ds(wid * _K, _K)], ewv)

    @pl.loop(0, _K // _NBUF)
    def _(i):
        j0 = i * _NBUF
        hs = []
        for b in range(_NBUF):
            _fill_rows(rows[b], ewv.at[j0 + b])
            hs.append(pltpu.async_copy(rows[b], acc.at[dstv.at[j0 + b]],
                                       ssem[b], add=True))
        for h in hs:
            h.wait()

    plsc.subcore_barrier()
    pltpu.sync_copy(acc.at[pl.ds(s * _NPS, _NPS)],
                    out_hbm.at[c, pl.ds(s * _NPS, _NPS)])


def _mm1_body(x_ref, w_ref, o_ref):
    o_ref[...] = jnp.dot(x_ref[...], w_ref[...],
                         preferred_element_type=jnp.float32)


_mm1 = pl.pallas_call(
    _mm1_body,
    out_shape=jax.ShapeDtypeStruct((_NP, _DH), jnp.float32),
)


def _prep_body(degp_ref, h1_ref, h1d_ref, sl1_ref, dinv_ref):
    degp = degp_ref[...]
    deg = degp[0, :, 0:1] + degp[1, :, 0:1] + 1.0
    dv = lax.rsqrt(deg)
    h1 = h1_ref[...]
    h1d = h1 * dv
    h1d_ref[...] = h1d
    sl1_ref[...] = h1d * dv
    dinv_ref[...] = dv


_prep = pl.pallas_call(
    _prep_body,
    out_shape=(
        jax.ShapeDtypeStruct((_NP, _DH), jnp.float32),
        jax.ShapeDtypeStruct((_NP, _DH), jnp.float32),
        jax.ShapeDtypeStruct((_NP, 1), jnp.float32),
    ),
)


def _mid_body(aggp_ref, sl1_ref, dinv_ref, b1_ref, z1d_ref, slz_ref):
    aggp = aggp_ref[...]
    dv = dinv_ref[...]
    t = dv * (aggp[0] + aggp[1]) + sl1_ref[...] + b1_ref[...]
    z1 = jnp.maximum(t, 0.0)
    z1d = z1 * dv
    z1d_ref[...] = z1d
    slz_ref[...] = z1d * dv


_mid = pl.pallas_call(
    _mid_body,
    out_shape=(
        jax.ShapeDtypeStruct((_NP, _DH), jnp.float32),
        jax.ShapeDtypeStruct((_NP, _DH), jnp.float32),
    ),
)


def _fin_body(aggp_ref, slz_ref, dinv_ref, w2_ref, b2_ref, o_ref):
    aggp = aggp_ref[...]
    pre = dinv_ref[...] * (aggp[0] + aggp[1]) + slz_ref[...]
    o_ref[...] = jnp.dot(pre, w2_ref[...],
                         preferred_element_type=jnp.float32) + b2_ref[...]


_fin = pl.pallas_call(
    _fin_body,
    out_shape=jax.ShapeDtypeStruct((_NP, _DO), jnp.float32),
)


def kernel(x, edge_index, edge_weight, W1, b1, W2, b2):
    src = edge_index[0].astype(jnp.int32)
    dst = edge_index[1].astype(jnp.int32)
    ew = edge_weight.astype(jnp.float32)
    pad = _E_PAD - _E
    src3 = jnp.pad(src, (0, pad)).reshape(_NCHUNK, _C)
    dst3 = jnp.pad(dst, (0, pad)).reshape(_NCHUNK, _C)
    ew3 = jnp.pad(ew, (0, pad)).reshape(_NCHUNK, _C)

    xp = jnp.pad(x, ((0, _NP - _N), (0, 0)))
    h1 = _mm1(xp, W1)
    degp = _deg_sc(dst3, ew3)
    h1d, sl1, dinv = _prep(degp, h1)
    aggp1 = _agg_sc(h1d, src3, dst3, ew3)
    z1d, slz = _mid(aggp1, sl1, dinv, b1.reshape(1, _DH))
    aggp2 = _agg_sc(z1d, src3, dst3, ew3)
    out = _fin(aggp2, slz, dinv, W2, b2.reshape(1, _DO))
    return out[:_N]


# speedup vs baseline: 47.7345x; 1.3327x over previous
import functools

import jax
import jax.numpy as jnp
from jax import lax
from jax.experimental import pallas as pl
from jax.experimental.pallas import tpu as pltpu
from jax.experimental.pallas import tpu_sc as plsc

_N = 10000
_E = 320000
_DIN = 128
_DH = 16
_DO = 64

_NC = 2
_NS = 16
_NW = _NC * _NS
_C = 128
_K = 80
_EPT = _K * _C
_E_PAD = _NW * _EPT
_NP = 10240
_NPS = _NP // _NS

_mesh = plsc.VectorSubcoreMesh(core_axis_name="c", subcore_axis_name="s")

_sc_params = pltpu.CompilerParams(needs_layout_passes=False,
                                  use_tc_tiling_on_sc=False)


def _zero_acc(zbuf, acc, s):
    @pl.loop(0, _NPS)
    def _(i):
        zbuf.at[i][...] = jnp.zeros((16,), jnp.float32)

    pltpu.sync_copy(zbuf, acc.at[pl.ds(s * _NPS, _NPS)])


_NBUF = 8
_NCHUNK = _E_PAD // _C
_KC0 = 80
_KC1 = 80
_KMAX = max(_KC0, _KC1)


def _scale_rows(rows, ewrow):
    @pl.loop(0, _C // 16)
    def _(g):
        base = g * 16
        ev = ewrow[pl.ds(base, 16)]
        for t in range(16):
            e = base + t
            splat = jnp.take(ev, jnp.full((16,), t, jnp.int32))
            rows.at[e][...] = rows.at[e][...] * splat


def _fill_rows(rows, ewrow):
    @pl.loop(0, _C // 16)
    def _(g):
        base = g * 16
        ev = ewrow[pl.ds(base, 16)]
        for t in range(16):
            e = base + t
            rows.at[e][...] = jnp.take(ev, jnp.full((16,), t, jnp.int32))


@functools.partial(
    pl.kernel,
    out_type=jax.ShapeDtypeStruct((_NC, _NP, _DH), jnp.float32),
    mesh=_mesh,
    compiler_params=_sc_params,
    scratch_types=(
        [
            pltpu.VMEM((_KMAX, _C), jnp.int32),
            pltpu.VMEM((_KMAX, _C), jnp.int32),
            pltpu.VMEM((_KMAX, _C), jnp.float32),
        ]
        + [pltpu.VMEM((_C, _DH), jnp.float32) for _ in range(_NBUF)]
        + [pltpu.VMEM((_NPS, _DH), jnp.float32),
           pltpu.VMEM_SHARED((_NP, _DH), jnp.float32),
           pltpu.VMEM_SHARED((_NP, _DH), jnp.float32)]
        + [pltpu.SemaphoreType.DMA for _ in range(2 * _NBUF)]
    ),
)
def _agg_sc(h_hbm, src_hbm, dst_hbm, ew_hbm, out_hbm,
            srcv, dstv, ewv, *rest):
    rows = rest[:_NBUF]
    zbuf, acc, htab = rest[_NBUF], rest[_NBUF + 1], rest[_NBUF + 2]
    gsem = rest[_NBUF + 3:_NBUF + 3 + _NBUF]
    ssem = rest[_NBUF + 3 + _NBUF:]
    c = lax.axis_index("c")
    s = lax.axis_index("s")

    _zero_acc(zbuf, acc, s)

    @pl.when(s == 0)
    def _():
        pltpu.sync_copy(h_hbm, htab)

    plsc.subcore_barrier()

    def _edges(off, nk):
        pltpu.sync_copy(src_hbm.at[pl.ds(off, nk)], srcv.at[pl.ds(0, nk)])
        pltpu.sync_copy(dst_hbm.at[pl.ds(off, nk)], dstv.at[pl.ds(0, nk)])
        pltpu.sync_copy(ew_hbm.at[pl.ds(off, nk)], ewv.at[pl.ds(0, nk)])

        @pl.loop(0, nk // _NBUF)
        def _(i):
            j0 = i * _NBUF
            hg = [pltpu.async_copy(htab.at[srcv.at[j0 + b]], rows[b],
                                   gsem[b])
                  for b in range(_NBUF)]
            hs = []
            for b in range(_NBUF):
                hg[b].wait()
                _scale_rows(rows[b], ewv.at[j0 + b])
                hs.append(pltpu.async_copy(rows[b], acc.at[dstv.at[j0 + b]],
                                           ssem[b], add=True))
            for h in hs:
                h.wait()

    @pl.when(c == 0)
    def _():
        _edges(s * _KC0, _KC0)

    @pl.when(c == 1)
    def _():
        _edges(_NS * _KC0 + s * _KC1, _KC1)

    plsc.subcore_barrier()
    pltpu.sync_copy(acc.at[pl.ds(s * _NPS, _NPS)],
                    out_hbm.at[c, pl.ds(s * _NPS, _NPS)])


@functools.partial(
    pl.kernel,
    out_type=jax.ShapeDtypeStruct((_NC, _NP, _DH), jnp.float32),
    mesh=_mesh,
    compiler_params=_sc_params,
    scratch_types=(
        [
            pltpu.VMEM((_K, _C), jnp.int32),
            pltpu.VMEM((_K, _C), jnp.float32),
        ]
        + [pltpu.VMEM((_C, _DH), jnp.float32) for _ in range(_NBUF)]
        + [pltpu.VMEM((_NPS, _DH), jnp.float32),
           pltpu.VMEM_SHARED((_NP, _DH), jnp.float32)]
        + [pltpu.SemaphoreType.DMA for _ in range(_NBUF)]
    ),
)
def _deg_sc(dst_hbm, ew_hbm, out_hbm, dstv, ewv, *rest):
    rows = rest[:_NBUF]
    zbuf, acc = rest[_NBUF], rest[_NBUF + 1]
    ssem = rest[_NBUF + 2:]
    c = lax.axis_index("c")
    s = lax.axis_index("s")
    wid = c * _NS + s

    _zero_acc(zbuf, acc, s)
    plsc.subcore_barrier()

    pltpu.sync_copy(dst_hbm.at[pl.ds(wid * _K, _K)], dstv)
    pltpu.sync_copy(ew_hbm.at[pl.ds(wid * _K, _K)], ewv)

    @pl.loop(0, _K // _NBUF)
    def _(i):
        j0 = i * _NBUF
        hs = []
        for b in range(_NBUF):
            _fill_rows(rows[b], ewv.at[j0 + b])
            hs.append(pltpu.async_copy(rows[b], acc.at[dstv.at[j0 + b]],
                                       ssem[b], add=True))
        for h in hs:
            h.wait()

    plsc.subcore_barrier()
    pltpu.sync_copy(acc.at[pl.ds(s * _NPS, _NPS)],
                    out_hbm.at[c, pl.ds(s * _NPS, _NPS)])


def _mm1_body(x_ref, w_ref, o_ref):
    o_ref[...] = jnp.dot(x_ref[...], w_ref[...],
                         preferred_element_type=jnp.float32)


_mm1 = pl.pallas_call(
    _mm1_body,
    out_shape=jax.ShapeDtypeStruct((_NP, _DH), jnp.float32),
)


def _prep_body(degp_ref, h1_ref, h1d_ref, sl1_ref, dinv_ref):
    degp = degp_ref[...]
    deg = degp[0, :, 0:1] + degp[1, :, 0:1] + 1.0
    dv = lax.rsqrt(deg)
    h1 = h1_ref[...]
    h1d = h1 * dv
    h1d_ref[...] = h1d
    sl1_ref[...] = h1d * dv
    dinv_ref[...] = dv


_prep = pl.pallas_call(
    _prep_body,
    out_shape=(
        jax.ShapeDtypeStruct((_NP, _DH), jnp.float32),
        jax.ShapeDtypeStruct((_NP, _DH), jnp.float32),
        jax.ShapeDtypeStruct((_NP, 1), jnp.float32),
    ),
)


def _mid_body(aggp_ref, sl1_ref, dinv_ref, b1_ref, z1d_ref, slz_ref):
    aggp = aggp_ref[...]
    dv = dinv_ref[...]
    t = dv * (aggp[0] + aggp[1]) + sl1_ref[...] + b1_ref[...]
    z1 = jnp.maximum(t, 0.0)
    z1d = z1 * dv
    z1d_ref[...] = z1d
    slz_ref[...] = z1d * dv


_mid = pl.pallas_call(
    _mid_body,
    out_shape=(
        jax.ShapeDtypeStruct((_NP, _DH), jnp.float32),
        jax.ShapeDtypeStruct((_NP, _DH), jnp.float32),
    ),
)


def _fin_body(aggp_ref, slz_ref, dinv_ref, w2_ref, b2_ref, o_ref):
    aggp = aggp_ref[...]
    pre = dinv_ref[...] * (aggp[0] + aggp[1]) + slz_ref[...]
    o_ref[...] = jnp.dot(pre, w2_ref[...],
                         preferred_element_type=jnp.float32) + b2_ref[...]


_fin = pl.pallas_call(
    _fin_body,
    out_shape=jax.ShapeDtypeStruct((_NP, _DO), jnp.float32),
)


def kernel(x, edge_index, edge_weight, W1, b1, W2, b2):
    src = edge_index[0].astype(jnp.int32)
    dst = edge_index[1].astype(jnp.int32)
    ew = edge_weight.astype(jnp.float32)
    pad = _E_PAD - _E
    src3 = jnp.pad(src, (0, pad)).reshape(_NCHUNK, _C)
    dst3 = jnp.pad(dst, (0, pad)).reshape(_NCHUNK, _C)
    ew3 = jnp.pad(ew, (0, pad)).reshape(_NCHUNK, _C)

    xp = jnp.pad(x, ((0, _NP - _N), (0, 0)))
    h1 = _mm1(xp, W1)
    degp = _deg_sc(dst3, ew3)
    h1d, sl1, dinv = _prep(degp, h1)
    aggp1 = _agg_sc(h1d, src3, dst3, ew3)
    z1d, slz = _mid(aggp1, sl1, dinv, b1.reshape(1, _DH))
    aggp2 = _agg_sc(z1d, src3, dst3, ew3)
    out = _fin(aggp2, slz, dinv, W2, b2.reshape(1, _DO))
    return out[:_N]
